# baseline (speedup 1.0000x reference)
import jax
import jax.numpy as jnp
from jax import lax
from jax.experimental import pallas as pl
from jax.experimental.pallas import tpu as pltpu
from jax.experimental.pallas import tpu_sc as plsc

DIM = 512
CODE_SIZE = 16384
CODE_DEPTH = 8
MAX_LEN = 2048
B = 16
T = MAX_LEN - 1
V = CODE_SIZE + 2
HALF = MAX_LEN // 2
CH = 32
NCH = HALF // CH
TOKROW = 16384


def _body(tokens_hbm, lengths_hbm, table_hbm, eot_hbm, eot2_hbm,
          out1_hbm, out2_hbm, mask_hbm,
          tokbuf, idxbuf, acc, b0, b1, t0, lenbuf, e1buf, e2buf, maskbuf,
          sem, semt, sema, semb0, semb1):
    cid = lax.axis_index("c")
    sid = lax.axis_index("s")
    wid = sid * 2 + cid
    b = wid // 2
    h = wid % 2
    row_base = b * MAX_LEN + h * HALF
    lanes = lax.iota(jnp.int32, 16)

    pltpu.sync_copy(tokens_hbm.at[b], tokbuf)
    pltpu.sync_copy(eot_hbm, e1buf)
    pltpu.sync_copy(eot2_hbm, e2buf)

    lenbuf[0, pl.ds(0, 16)] = jnp.full((16,), b, jnp.int32)
    pltpu.async_copy(lengths_hbm.at[lenbuf.at[0]], lenbuf.at[1], sem).wait()
    len2v = jnp.minimum(lenbuf[1, pl.ds(0, 16)] + jnp.full((16,), 1, jnp.int32),
                        jnp.full((16,), MAX_LEN, jnp.int32))

    def mask_body(j, carry):
        pos = h * HALF + j * 16
        posv = lanes + jnp.full((16,), pos, jnp.int32)
        diff = len2v - posv
        zero = jnp.full((16,), 0, jnp.int32)
        one = jnp.full((16,), 1, jnp.int32)
        maskbuf[pl.ds(j * 16, 16)] = jnp.minimum(jnp.maximum(diff, zero), one)
        return carry

    lax.fori_loop(0, HALF // 16, mask_body, 0)
    pltpu.sync_copy(maskbuf, mask_hbm.at[pl.ds(row_base, HALF)])

    def chunk_body(c, carry):
        pos0 = c * CH
        for k in range(CODE_DEPTH):
            for j in range(CH // 16):
                off = k * T + h * HALF + pos0 + j * 16
                v = tokbuf[pl.ds(off, 16)] + jnp.full((16,), k * V, jnp.int32)
                idxbuf[k, pl.ds(j * 16, 16)] = v

        first_sc = (1 - h) * (1 - jnp.minimum(c, 1))
        indf = jnp.full((16,), first_sc.astype(jnp.float32), jnp.float32)

        cp_t = pltpu.async_copy(table_hbm.at[idxbuf.at[0]], t0, semt)
        cp_a = pltpu.async_copy(table_hbm.at[idxbuf.at[1]], acc, sema)
        bufs = (b0, b1)
        sems = (semb0, semb1)
        cps = [pltpu.async_copy(table_hbm.at[idxbuf.at[2]], b0, semb0), None]
        cp_a.wait()
        for k in range(2, CODE_DEPTH):
            cur = k % 2
            if k < CODE_DEPTH - 1:
                nxt = (k + 1) % 2
                cps[nxt] = pltpu.async_copy(
                    table_hbm.at[idxbuf.at[k + 1]], bufs[nxt], sems[nxt])
            cps[cur].wait()
            t = bufs[cur]

            def add_row(r, inner):
                a = acc.at[r]
                tt = t.at[r]
                for q in range(DIM // 16):
                    sl = pl.ds(q * 16, 16)
                    plsc.addupdate(a.at[sl], tt[sl])
                return inner

            lax.fori_loop(0, CH, add_row, 0)
        for q in range(DIM // 16):
            sl = pl.ds(q * 16, 16)
            a0 = acc[0, sl]
            acc[0, sl] = a0 + indf * (e2buf[sl] - a0)
        cp_t.wait()
        for q in range(DIM // 16):
            sl = pl.ds(q * 16, 16)
            t0v = t0[0, sl]
            t0[0, sl] = t0v + indf * (e1buf[sl] - t0v)
        pltpu.sync_copy(t0, out1_hbm.at[pl.ds(row_base + pos0, CH)])
        pltpu.sync_copy(acc, out2_hbm.at[pl.ds(row_base + pos0, CH)])
        return carry

    lax.fori_loop(0, NCH, chunk_body, 0)


def kernel(tokens, lengths, emb, EOT_emb, layer2_EOT_emb):
    table = emb.reshape(CODE_DEPTH * V, DIM)
    tokens_p = jnp.pad(tokens, ((0, 0), (1, TOKROW - CODE_DEPTH * T - 1)))
    mesh = plsc.VectorSubcoreMesh(core_axis_name="c", subcore_axis_name="s")
    out1, out2, mask = pl.kernel(
        _body,
        out_type=(
            jax.ShapeDtypeStruct((B * MAX_LEN, DIM), jnp.float32),
            jax.ShapeDtypeStruct((B * MAX_LEN, DIM), jnp.float32),
            jax.ShapeDtypeStruct((B * MAX_LEN,), jnp.int32),
        ),
        mesh=mesh,
        scratch_types=[
            pltpu.VMEM((TOKROW,), jnp.int32),
            pltpu.VMEM((CODE_DEPTH, CH), jnp.int32),
            pltpu.VMEM((CH, DIM), jnp.float32),
            pltpu.VMEM((CH, DIM), jnp.float32),
            pltpu.VMEM((CH, DIM), jnp.float32),
            pltpu.VMEM((CH, DIM), jnp.float32),
            pltpu.VMEM((2, 16), jnp.int32),
            pltpu.VMEM((DIM,), jnp.float32),
            pltpu.VMEM((DIM,), jnp.float32),
            pltpu.VMEM((HALF,), jnp.int32),
            pltpu.SemaphoreType.DMA,
            pltpu.SemaphoreType.DMA,
            pltpu.SemaphoreType.DMA,
            pltpu.SemaphoreType.DMA,
            pltpu.SemaphoreType.DMA,
        ],
    )(tokens_p, lengths, table, EOT_emb.reshape(DIM), layer2_EOT_emb.reshape(DIM))
    return (out1.reshape(B, MAX_LEN, DIM),
            out2.reshape(B, MAX_LEN, DIM),
            mask.reshape(B, MAX_LEN))


# speedup vs baseline: 1.4351x; 1.4351x over previous
import jax
import jax.numpy as jnp
from jax import lax
from jax.experimental import pallas as pl
from jax.experimental.pallas import tpu as pltpu
from jax.experimental.pallas import tpu_sc as plsc

DIM = 512
CODE_SIZE = 16384
CODE_DEPTH = 8
MAX_LEN = 2048
B = 16
T = MAX_LEN - 1
V = CODE_SIZE + 2
HALF = MAX_LEN // 2
CH = 64
NCH = HALF // CH
TOKROW = 16384


def _body(tokens_hbm, lengths_hbm, table_hbm, eot_hbm, eot2_hbm,
          out1_hbm, out2_hbm, mask_hbm,
          tokbuf, idxbuf, acc, b0, t0, lenbuf, e1buf, e2buf, maskbuf,
          sem, semt, sema, semb0):
    cid = lax.axis_index("c")
    sid = lax.axis_index("s")
    wid = sid * 2 + cid
    b = wid // 2
    h = wid % 2
    row_base = b * MAX_LEN + h * HALF
    lanes = lax.iota(jnp.int32, 16)

    pltpu.sync_copy(tokens_hbm.at[b], tokbuf)
    pltpu.sync_copy(eot_hbm, e1buf)
    pltpu.sync_copy(eot2_hbm, e2buf)

    lenbuf[0, pl.ds(0, 16)] = jnp.full((16,), b, jnp.int32)
    pltpu.async_copy(lengths_hbm.at[lenbuf.at[0]], lenbuf.at[1], sem).wait()
    len2v = jnp.minimum(lenbuf[1, pl.ds(0, 16)] + jnp.full((16,), 1, jnp.int32),
                        jnp.full((16,), MAX_LEN, jnp.int32))

    def mask_body(j, carry):
        pos = h * HALF + j * 16
        posv = lanes + jnp.full((16,), pos, jnp.int32)
        diff = len2v - posv
        zero = jnp.full((16,), 0, jnp.int32)
        one = jnp.full((16,), 1, jnp.int32)
        maskbuf[pl.ds(j * 16, 16)] = jnp.minimum(jnp.maximum(diff, zero), one)
        return carry

    lax.fori_loop(0, HALF // 16, mask_body, 0)
    pltpu.sync_copy(maskbuf, mask_hbm.at[pl.ds(row_base, HALF)])

    def chunk_body(c, carry):
        pos0 = c * CH
        for k in range(CODE_DEPTH):
            for j in range(CH // 16):
                off = k * T + h * HALF + pos0 + j * 16
                v = tokbuf[pl.ds(off, 16)] + jnp.full((16,), k * V, jnp.int32)
                idxbuf[k, pl.ds(j * 16, 16)] = v

        first_sc = (1 - h) * (1 - jnp.minimum(c, 1))
        indf = jnp.full((16,), first_sc.astype(jnp.float32), jnp.float32)

        cp_t = pltpu.async_copy(table_hbm.at[idxbuf.at[0]], t0, semt)
        cp_a = pltpu.async_copy(table_hbm.at[idxbuf.at[1]], acc, sema)
        bufs = (b0, t0)
        sems = (semb0, semt)
        cps = [pltpu.async_copy(table_hbm.at[idxbuf.at[2]], b0, semb0), None]

        cp_t.wait()
        for q in range(DIM // 16):
            sl = pl.ds(q * 16, 16)
            t0v = t0[0, sl]
            t0[0, sl] = t0v + indf * (e1buf[sl] - t0v)
        pltpu.sync_copy(t0, out1_hbm.at[pl.ds(row_base + pos0, CH)])
        cps[1] = pltpu.async_copy(table_hbm.at[idxbuf.at[3]], t0, semt)

        cp_a.wait()
        for k in range(2, CODE_DEPTH):
            cur = k % 2
            cps[cur].wait()
            t = bufs[cur]

            def add_row(r, inner):
                a = acc.at[r]
                tt = t.at[r]
                for q in range(DIM // 16):
                    sl = pl.ds(q * 16, 16)
                    a[sl] = a[sl] + tt[sl]
                return inner

            lax.fori_loop(0, CH, add_row, 0)
            if k + 2 < CODE_DEPTH:
                cps[cur] = pltpu.async_copy(
                    table_hbm.at[idxbuf.at[k + 2]], bufs[cur], sems[cur])
        for q in range(DIM // 16):
            sl = pl.ds(q * 16, 16)
            a0 = acc[0, sl]
            acc[0, sl] = a0 + indf * (e2buf[sl] - a0)
        pltpu.sync_copy(acc, out2_hbm.at[pl.ds(row_base + pos0, CH)])
        return carry

    lax.fori_loop(0, NCH, chunk_body, 0)


def kernel(tokens, lengths, emb, EOT_emb, layer2_EOT_emb):
    table = emb.reshape(CODE_DEPTH * V, DIM)
    tokens_p = jnp.pad(tokens, ((0, 0), (1, TOKROW - CODE_DEPTH * T - 1)))
    mesh = plsc.VectorSubcoreMesh(core_axis_name="c", subcore_axis_name="s")
    out1, out2, mask = pl.kernel(
        _body,
        out_type=(
            jax.ShapeDtypeStruct((B * MAX_LEN, DIM), jnp.float32),
            jax.ShapeDtypeStruct((B * MAX_LEN, DIM), jnp.float32),
            jax.ShapeDtypeStruct((B * MAX_LEN,), jnp.int32),
        ),
        mesh=mesh,
        scratch_types=[
            pltpu.VMEM((TOKROW,), jnp.int32),
            pltpu.VMEM((CODE_DEPTH, CH), jnp.int32),
            pltpu.VMEM((CH, DIM), jnp.float32),
            pltpu.VMEM((CH, DIM), jnp.float32),
            pltpu.VMEM((CH, DIM), jnp.float32),
            pltpu.VMEM((2, 16), jnp.int32),
            pltpu.VMEM((DIM,), jnp.float32),
            pltpu.VMEM((DIM,), jnp.float32),
            pltpu.VMEM((HALF,), jnp.int32),
            pltpu.SemaphoreType.DMA,
            pltpu.SemaphoreType.DMA,
            pltpu.SemaphoreType.DMA,
            pltpu.SemaphoreType.DMA,
        ],
    )(tokens_p, lengths, table, EOT_emb.reshape(DIM), layer2_EOT_emb.reshape(DIM))
    return (out1.reshape(B, MAX_LEN, DIM),
            out2.reshape(B, MAX_LEN, DIM),
            mask.reshape(B, MAX_LEN))


# speedup vs baseline: 2.0291x; 1.4140x over previous
import jax
import jax.numpy as jnp
from jax import lax
from jax.experimental import pallas as pl
from jax.experimental.pallas import tpu as pltpu
from jax.experimental.pallas import tpu_sc as plsc

DIM = 512
CODE_SIZE = 16384
CODE_DEPTH = 8
MAX_LEN = 2048
B = 16
T = MAX_LEN - 1
V = CODE_SIZE + 2
HALF = MAX_LEN // 2
CH = 64
NCH = HALF // CH
TOKROW = 16384


def _body(tokens_hbm, lengths_hbm, table_hbm, eot_hbm, eot2_hbm,
          out1_hbm, out2_hbm, mask_hbm,
          tokbuf, idxbuf, acc, b0, t0, lenbuf, e1buf, e2buf, maskbuf,
          sem, semt, sema, semb0):
    cid = lax.axis_index("c")
    sid = lax.axis_index("s")
    wid = sid * 2 + cid
    b = wid // 2
    h = wid % 2
    row_base = b * MAX_LEN + h * HALF
    lanes = lax.iota(jnp.int32, 16)

    pltpu.sync_copy(tokens_hbm.at[b], tokbuf)
    pltpu.sync_copy(eot_hbm, e1buf)
    pltpu.sync_copy(eot2_hbm, e2buf)

    lenbuf[0, pl.ds(0, 16)] = jnp.full((16,), b, jnp.int32)
    pltpu.async_copy(lengths_hbm.at[lenbuf.at[0]], lenbuf.at[1], sem).wait()
    len2v = jnp.minimum(lenbuf[1, pl.ds(0, 16)] + jnp.full((16,), 1, jnp.int32),
                        jnp.full((16,), MAX_LEN, jnp.int32))

    def mask_body(j, carry):
        pos = h * HALF + j * 16
        posv = lanes + jnp.full((16,), pos, jnp.int32)
        diff = len2v - posv
        zero = jnp.full((16,), 0, jnp.int32)
        one = jnp.full((16,), 1, jnp.int32)
        maskbuf[pl.ds(j * 16, 16)] = jnp.minimum(jnp.maximum(diff, zero), one)
        return carry

    lax.fori_loop(0, HALF // 16, mask_body, 0)
    pltpu.sync_copy(maskbuf, mask_hbm.at[pl.ds(row_base, HALF)])

    def chunk_body(c, carry):
        pos0 = c * CH
        for k in range(CODE_DEPTH):
            for j in range(CH // 16):
                off = k * T + h * HALF + pos0 + j * 16
                idxbuf[k, pl.ds(j * 16, 16)] = tokbuf[pl.ds(off, 16)]

        first_sc = (1 - h) * (1 - jnp.minimum(c, 1))
        indf = jnp.full((16,), first_sc.astype(jnp.float32), jnp.float32)

        cp_t = pltpu.async_copy(table_hbm.at[0].at[idxbuf.at[0]], t0, semt)
        cp_a = pltpu.async_copy(table_hbm.at[1].at[idxbuf.at[1]], acc, sema)
        bufs = (b0, t0)
        sems = (semb0, semt)
        cps = [pltpu.async_copy(table_hbm.at[2].at[idxbuf.at[2]], b0, semb0), None]

        cp_t.wait()
        for q in range(DIM // 16):
            sl = pl.ds(q * 16, 16)
            t0v = t0[0, sl]
            t0[0, sl] = t0v + indf * (e1buf[sl] - t0v)
        pltpu.sync_copy(t0, out1_hbm.at[pl.ds(row_base + pos0, CH)])
        cps[1] = pltpu.async_copy(table_hbm.at[3].at[idxbuf.at[3]], t0, semt)

        cp_a.wait()
        for k in range(2, CODE_DEPTH):
            cur = k % 2
            cps[cur].wait()
            t = bufs[cur]

            def add_row(r, inner):
                a = acc.at[r]
                tt = t.at[r]
                for q in range(DIM // 16):
                    sl = pl.ds(q * 16, 16)
                    a[sl] = a[sl] + tt[sl]
                return inner

            lax.fori_loop(0, CH, add_row, 0)
            if k + 2 < CODE_DEPTH:
                cps[cur] = pltpu.async_copy(
                    table_hbm.at[k + 2].at[idxbuf.at[k + 2]], bufs[cur],
                    sems[cur])
        for q in range(DIM // 16):
            sl = pl.ds(q * 16, 16)
            a0 = acc[0, sl]
            acc[0, sl] = a0 + indf * (e2buf[sl] - a0)
        pltpu.sync_copy(acc, out2_hbm.at[pl.ds(row_base + pos0, CH)])
        return carry

    lax.fori_loop(0, NCH, chunk_body, 0)


def kernel(tokens, lengths, emb, EOT_emb, layer2_EOT_emb):
    tokens_p = jnp.pad(tokens, ((0, 0), (1, TOKROW - CODE_DEPTH * T - 1)))
    mesh = plsc.VectorSubcoreMesh(core_axis_name="c", subcore_axis_name="s")
    out1, out2, mask = pl.kernel(
        _body,
        out_type=(
            jax.ShapeDtypeStruct((B * MAX_LEN, DIM), jnp.float32),
            jax.ShapeDtypeStruct((B * MAX_LEN, DIM), jnp.float32),
            jax.ShapeDtypeStruct((B * MAX_LEN,), jnp.int32),
        ),
        mesh=mesh,
        scratch_types=[
            pltpu.VMEM((TOKROW,), jnp.int32),
            pltpu.VMEM((CODE_DEPTH, CH), jnp.int32),
            pltpu.VMEM((CH, DIM), jnp.float32),
            pltpu.VMEM((CH, DIM), jnp.float32),
            pltpu.VMEM((CH, DIM), jnp.float32),
            pltpu.VMEM((2, 16), jnp.int32),
            pltpu.VMEM((DIM,), jnp.float32),
            pltpu.VMEM((DIM,), jnp.float32),
            pltpu.VMEM((HALF,), jnp.int32),
            pltpu.SemaphoreType.DMA,
            pltpu.SemaphoreType.DMA,
            pltpu.SemaphoreType.DMA,
            pltpu.SemaphoreType.DMA,
        ],
    )(tokens_p, lengths, emb, EOT_emb.reshape(DIM), layer2_EOT_emb.reshape(DIM))
    return (out1.reshape(B, MAX_LEN, DIM),
            out2.reshape(B, MAX_LEN, DIM),
            mask.reshape(B, MAX_LEN))


# speedup vs baseline: 2.0364x; 1.0036x over previous
import jax
import jax.numpy as jnp
from jax import lax
from jax.experimental import pallas as pl
from jax.experimental.pallas import tpu as pltpu
from jax.experimental.pallas import tpu_sc as plsc

DIM = 512
CODE_SIZE = 16384
CODE_DEPTH = 8
MAX_LEN = 2048
B = 16
T = MAX_LEN - 1
V = CODE_SIZE + 2
HALF = MAX_LEN // 2
CH = 64
NCH = HALF // CH
TOKROW = 16384


def _body(tokens_hbm, lengths_hbm, table_hbm, eot_hbm, eot2_hbm,
          out1_hbm, out2_hbm, mask_hbm,
          tokbuf, idxbuf, acc, b0, t0, lenbuf, e1buf, e2buf, maskbuf,
          sem, semt, sema, semb0):
    cid = lax.axis_index("c")
    sid = lax.axis_index("s")
    wid = sid * 2 + cid
    b = wid // 2
    h = wid % 2
    row0 = h * HALF
    lanes = lax.iota(jnp.int32, 16)

    pltpu.sync_copy(tokens_hbm.at[b], tokbuf)
    pltpu.sync_copy(eot_hbm, e1buf)
    pltpu.sync_copy(eot2_hbm, e2buf)

    lenbuf[0, pl.ds(0, 16)] = jnp.full((16,), b, jnp.int32)
    pltpu.async_copy(lengths_hbm.at[lenbuf.at[0]], lenbuf.at[1], sem).wait()
    len2v = jnp.minimum(lenbuf[1, pl.ds(0, 16)] + jnp.full((16,), 1, jnp.int32),
                        jnp.full((16,), MAX_LEN, jnp.int32))

    def mask_body(j, carry):
        pos = h * HALF + j * 16
        posv = lanes + jnp.full((16,), pos, jnp.int32)
        diff = len2v - posv
        zero = jnp.full((16,), 0, jnp.int32)
        one = jnp.full((16,), 1, jnp.int32)
        maskbuf[pl.ds(j * 16, 16)] = jnp.minimum(jnp.maximum(diff, zero), one)
        return carry

    lax.fori_loop(0, HALF // 16, mask_body, 0)
    pltpu.sync_copy(maskbuf, mask_hbm.at[b].at[pl.ds(row0, HALF)])

    def chunk_body(c, carry):
        pos0 = c * CH
        for k in range(CODE_DEPTH):
            for j in range(CH // 16):
                off = k * T + h * HALF + pos0 + j * 16
                idxbuf[k, pl.ds(j * 16, 16)] = tokbuf[pl.ds(off, 16)]

        first_sc = (1 - h) * (1 - jnp.minimum(c, 1))
        indf = jnp.full((16,), first_sc.astype(jnp.float32), jnp.float32)

        cp_t = pltpu.async_copy(table_hbm.at[0].at[idxbuf.at[0]], t0, semt)
        cp_a = pltpu.async_copy(table_hbm.at[1].at[idxbuf.at[1]], acc, sema)
        bufs = (b0, t0)
        sems = (semb0, semt)
        cps = [pltpu.async_copy(table_hbm.at[2].at[idxbuf.at[2]], b0, semb0), None]

        cp_t.wait()
        for q in range(DIM // 16):
            sl = pl.ds(q * 16, 16)
            t0v = t0[0, sl]
            t0[0, sl] = t0v + indf * (e1buf[sl] - t0v)
        pltpu.sync_copy(t0, out1_hbm.at[b].at[pl.ds(row0 + pos0, CH)])
        cps[1] = pltpu.async_copy(table_hbm.at[3].at[idxbuf.at[3]], t0, semt)

        cp_a.wait()
        for k in range(2, CODE_DEPTH):
            cur = k % 2
            cps[cur].wait()
            t = bufs[cur]

            def add_row(r, inner):
                a = acc.at[r]
                tt = t.at[r]
                for q in range(DIM // 16):
                    sl = pl.ds(q * 16, 16)
                    a[sl] = a[sl] + tt[sl]
                return inner

            lax.fori_loop(0, CH, add_row, 0)
            if k + 2 < CODE_DEPTH:
                cps[cur] = pltpu.async_copy(
                    table_hbm.at[k + 2].at[idxbuf.at[k + 2]], bufs[cur],
                    sems[cur])
        for q in range(DIM // 16):
            sl = pl.ds(q * 16, 16)
            a0 = acc[0, sl]
            acc[0, sl] = a0 + indf * (e2buf[sl] - a0)
        pltpu.sync_copy(acc, out2_hbm.at[b].at[pl.ds(row0 + pos0, CH)])
        return carry

    lax.fori_loop(0, NCH, chunk_body, 0)


def kernel(tokens, lengths, emb, EOT_emb, layer2_EOT_emb):
    tokens_p = jnp.pad(tokens, ((0, 0), (1, TOKROW - CODE_DEPTH * T - 1)))
    mesh = plsc.VectorSubcoreMesh(core_axis_name="c", subcore_axis_name="s")
    out1, out2, mask = pl.kernel(
        _body,
        out_type=(
            jax.ShapeDtypeStruct((B, MAX_LEN, DIM), jnp.float32),
            jax.ShapeDtypeStruct((B, MAX_LEN, DIM), jnp.float32),
            jax.ShapeDtypeStruct((B, MAX_LEN), jnp.int32),
        ),
        mesh=mesh,
        scratch_types=[
            pltpu.VMEM((TOKROW,), jnp.int32),
            pltpu.VMEM((CODE_DEPTH, CH), jnp.int32),
            pltpu.VMEM((CH, DIM), jnp.float32),
            pltpu.VMEM((CH, DIM), jnp.float32),
            pltpu.VMEM((CH, DIM), jnp.float32),
            pltpu.VMEM((2, 16), jnp.int32),
            pltpu.VMEM((DIM,), jnp.float32),
            pltpu.VMEM((DIM,), jnp.float32),
            pltpu.VMEM((HALF,), jnp.int32),
            pltpu.SemaphoreType.DMA,
            pltpu.SemaphoreType.DMA,
            pltpu.SemaphoreType.DMA,
            pltpu.SemaphoreType.DMA,
        ],
    )(tokens_p, lengths, emb, EOT_emb.reshape(DIM), layer2_EOT_emb.reshape(DIM))
    return (out1, out2, mask)


# speedup vs baseline: 2.0367x; 1.0002x over previous
import jax
import jax.numpy as jnp
from jax import lax
from jax.experimental import pallas as pl
from jax.experimental.pallas import tpu as pltpu
from jax.experimental.pallas import tpu_sc as plsc

DIM = 512
CODE_SIZE = 16384
CODE_DEPTH = 8
MAX_LEN = 2048
B = 16
T = MAX_LEN - 1
V = CODE_SIZE + 2
HALF = MAX_LEN // 2
CH = 64
NCH = HALF // CH
TOKROW = 16384


def _body(tokens_hbm, lengths_hbm, table_hbm, eot_hbm, eot2_hbm,
          out1_hbm, out2_hbm, mask_hbm,
          tokbuf, idxbuf, acc, b0, t0, lenbuf, e1buf, e2buf, maskbuf,
          sem, semt, sema, semb0):
    cid = lax.axis_index("c")
    sid = lax.axis_index("s")
    wid = sid * 2 + cid
    b = wid // 2
    h = wid % 2
    row0 = h * HALF
    lanes = lax.iota(jnp.int32, 16)

    pltpu.sync_copy(tokens_hbm.at[b], tokbuf)
    pltpu.sync_copy(eot_hbm, e1buf)
    pltpu.sync_copy(eot2_hbm, e2buf)

    lenbuf[0, pl.ds(0, 16)] = jnp.full((16,), b, jnp.int32)
    pltpu.async_copy(lengths_hbm.at[lenbuf.at[0]], lenbuf.at[1], sem).wait()
    len2v = jnp.minimum(lenbuf[1, pl.ds(0, 16)] + jnp.full((16,), 1, jnp.int32),
                        jnp.full((16,), MAX_LEN, jnp.int32))

    def mask_body(j, carry):
        pos = h * HALF + j * 16
        posv = lanes + jnp.full((16,), pos, jnp.int32)
        diff = len2v - posv
        zero = jnp.full((16,), 0, jnp.int32)
        one = jnp.full((16,), 1, jnp.int32)
        maskbuf[pl.ds(j * 16, 16)] = jnp.minimum(jnp.maximum(diff, zero), one)
        return carry

    lax.fori_loop(0, HALF // 16, mask_body, 0)
    pltpu.sync_copy(maskbuf, mask_hbm.at[b].at[pl.ds(row0, HALF)])

    def chunk_body(c, carry):
        pos0 = c * CH
        for k in range(CODE_DEPTH):
            for j in range(CH // 16):
                off = k * T + h * HALF + pos0 + j * 16
                idxbuf[k, pl.ds(j * 16, 16)] = tokbuf[pl.ds(off, 16)]

        first_sc = (1 - h) * (1 - jnp.minimum(c, 1))
        indf = jnp.full((16,), first_sc.astype(jnp.float32), jnp.float32)

        cp_t = pltpu.async_copy(table_hbm.at[0].at[idxbuf.at[0]], t0, semt)
        cp_a = pltpu.async_copy(table_hbm.at[1].at[idxbuf.at[1]], acc, sema)
        bufs = (b0, t0)
        sems = (semb0, semt)
        cps = [pltpu.async_copy(table_hbm.at[2].at[idxbuf.at[2]], b0, semb0), None]

        cp_t.wait()
        for q in range(DIM // 16):
            sl = pl.ds(q * 16, 16)
            t0v = t0[0, sl]
            t0[0, sl] = t0v + indf * (e1buf[sl] - t0v)
        pltpu.sync_copy(t0, out1_hbm.at[b].at[pl.ds(row0 + pos0, CH)])
        cps[1] = pltpu.async_copy(table_hbm.at[3].at[idxbuf.at[3]], t0, semt)

        cp_a.wait()
        for k in range(2, CODE_DEPTH):
            cur = k % 2
            cps[cur].wait()
            t = bufs[cur]

            def add_row(r, inner):
                a = acc.at[r]
                tt = t.at[r]
                for q in range(DIM // 16):
                    sl = pl.ds(q * 16, 16)
                    plsc.addupdate(a.at[pl.ds(q * 16, 16)], tt[sl])
                return inner

            lax.fori_loop(0, CH, add_row, 0)
            if k + 2 < CODE_DEPTH:
                cps[cur] = pltpu.async_copy(
                    table_hbm.at[k + 2].at[idxbuf.at[k + 2]], bufs[cur],
                    sems[cur])
        for q in range(DIM // 16):
            sl = pl.ds(q * 16, 16)
            a0 = acc[0, sl]
            acc[0, sl] = a0 + indf * (e2buf[sl] - a0)
        pltpu.sync_copy(acc, out2_hbm.at[b].at[pl.ds(row0 + pos0, CH)])
        return carry

    lax.fori_loop(0, NCH, chunk_body, 0)


def kernel(tokens, lengths, emb, EOT_emb, layer2_EOT_emb):
    tokens_p = jnp.pad(tokens, ((0, 0), (1, TOKROW - CODE_DEPTH * T - 1)))
    mesh = plsc.VectorSubcoreMesh(core_axis_name="c", subcore_axis_name="s")
    out1, out2, mask = pl.kernel(
        _body,
        out_type=(
            jax.ShapeDtypeStruct((B, MAX_LEN, DIM), jnp.float32),
            jax.ShapeDtypeStruct((B, MAX_LEN, DIM), jnp.float32),
            jax.ShapeDtypeStruct((B, MAX_LEN), jnp.int32),
        ),
        mesh=mesh,
        scratch_types=[
            pltpu.VMEM((TOKROW,), jnp.int32),
            pltpu.VMEM((CODE_DEPTH, CH), jnp.int32),
            pltpu.VMEM((CH, DIM), jnp.float32),
            pltpu.VMEM((CH, DIM), jnp.float32),
            pltpu.VMEM((CH, DIM), jnp.float32),
            pltpu.VMEM((2, 16), jnp.int32),
            pltpu.VMEM((DIM,), jnp.float32),
            pltpu.VMEM((DIM,), jnp.float32),
            pltpu.VMEM((HALF,), jnp.int32),
            pltpu.SemaphoreType.DMA,
            pltpu.SemaphoreType.DMA,
            pltpu.SemaphoreType.DMA,
            pltpu.SemaphoreType.DMA,
        ],
    )(tokens_p, lengths, emb, EOT_emb.reshape(DIM), layer2_EOT_emb.reshape(DIM))
    return (out1, out2, mask)
